# Initial kernel scaffold; baseline (speedup 1.0000x reference)
import jax
import jax.numpy as jnp
from jax.experimental import pallas as pl


def kernel(xyzs_0, xyzs_1, xyzs_2, l0_W1, l0_b1, l0_W2, l0_b2, m0_W1, m0_b1, m0_W2, m0_b2, c0_W, c0_b, m1_W1, m1_b1, m1_W2, m1_b2, c1_W, c1_b):
    raise NotImplementedError("write your pallas kernel here")


# speedup vs baseline: 4.2082x; 4.2082x over previous
import functools

import jax
import jax.numpy as jnp
from jax import lax
from jax.experimental import pallas as pl
from jax.experimental.pallas import tpu as pltpu
from jax.experimental.pallas import tpu_sc as plsc

N0, N1, N2 = 8192, 2048, 512
KNN = 16
L = 16
NC, NS = 2, 16
NW = NC * NS
QW1 = N1 // NW
QW2 = N2 // NW
C1 = N0 // L
C2 = N1 // L
O1, O2 = 32, 64


def _tc1_body(x0_r, x1_r, l0W1_r, l0b1_r, l0W2_r, l0b2_r,
              m0W1_r, m0b1_r, m0W2_r, m0b2_r, c0W_r, c0b_r,
              feat0_o, g1_o, bb1_o, sq0_o, sq1_o):
  x0 = x0_r[...]
  x1 = x1_r[...]
  dot = functools.partial(jnp.dot, preferred_element_type=jnp.float32)
  h = jnp.maximum(dot(l0W1_r[...], x0) + l0b1_r[...], 0.0)
  feat0 = jnp.maximum(dot(l0W2_r[...], h) + l0b2_r[...], 0.0)
  feat0_o[...] = feat0
  h = jnp.maximum(dot(m0W1_r[...], feat0) + m0b1_r[...], 0.0)
  f = jnp.maximum(dot(m0W2_r[...], h) + m0b2_r[...], 0.0)
  c0W = c0W_r[...]
  g1_o[...] = dot(c0W[:, :3], x0) + dot(c0W[:, 3:], f)
  bb1_o[...] = c0b_r[...] - dot(c0W[:, :3], x1)
  sq0_o[...] = jnp.sum(x0 * x0, axis=0, keepdims=True)
  sq1_o[...] = jnp.sum(x1 * x1, axis=0, keepdims=True)


def _tc2_body(feat1_r, x1_r, x2_r, m1W1_r, m1b1_r, m1W2_r, m1b2_r,
              c1W_r, c1b_r, g2_o, bb2_o):
  dot = functools.partial(jnp.dot, preferred_element_type=jnp.float32)
  feat1 = feat1_r[...]
  h = jnp.maximum(dot(m1W1_r[...], feat1) + m1b1_r[...], 0.0)
  f2 = jnp.maximum(dot(m1W2_r[...], h) + m1b2_r[...], 0.0)
  c1W = c1W_r[...]
  g2_o[...] = dot(c1W[:, :3], x1_r[...]) + dot(c1W[:, 3:], f2)
  bb2_o[...] = c1b_r[...] - dot(c1W[:, :3], x2_r[...])


def _knn_scan(qx, qy, qz, xv, yv, zv, sv, n_chunks):
  def chunk_step(c, carry):
    rd, ri = carry
    x = xv[pl.ds(c * L, L)]
    y = yv[pl.ds(c * L, L)]
    z = zv[pl.ds(c * L, L)]
    s = sv[pl.ds(c * L, L)]
    d = s - 2.0 * (qx * x + qy * y + qz * z)
    hit = d < rd[15]
    n = plsc.all_reduce_population_count(hit)[0]

    def do_merge():
      ji = lax.iota(jnp.int32, L) + c * L
      cd, ci = plsc.sort_key_val(d, ji, descending=True)
      m = cd < rd
      nd = jnp.where(m, cd, rd)
      ni = jnp.where(m, ci, ri)
      nd2, ni2 = plsc.sort_key_val(nd, ni)
      return (nd2, ni2)

    return lax.cond(n > 0, do_merge, lambda: (rd, ri))

  rd0 = jnp.full((L,), jnp.inf, jnp.float32)
  ri0 = jnp.zeros((L,), jnp.int32)
  return lax.fori_loop(0, n_chunks, chunk_step, (rd0, ri0))


def _make_sc1():
  mesh = plsc.VectorSubcoreMesh(core_axis_name="c", subcore_axis_name="s",
                                num_cores=NC, num_subcores=NS)

  @functools.partial(
      pl.kernel,
      out_type=[
          jax.ShapeDtypeStruct((N1, O1), jnp.float32),
          jax.ShapeDtypeStruct((N2, KNN), jnp.int32),
      ],
      mesh=mesh,
      compiler_params=pltpu.CompilerParams(
          needs_layout_passes=False, use_tc_tiling_on_sc=False),
      scratch_types=[
          pltpu.VMEM((N0,), jnp.float32),
          pltpu.VMEM((N0,), jnp.float32),
          pltpu.VMEM((N0,), jnp.float32),
          pltpu.VMEM((N0,), jnp.float32),
          pltpu.VMEM((QW1 * L,), jnp.float32),
          pltpu.VMEM((QW1 * L,), jnp.float32),
          pltpu.VMEM((QW1 * L,), jnp.float32),
          pltpu.VMEM((QW1, O1), jnp.float32),
          pltpu.VMEM((QW1 * KNN,), jnp.int32),
          pltpu.VMEM((QW1 * KNN, O1), jnp.float32),
          pltpu.VMEM((QW1, O1), jnp.float32),
          pltpu.VMEM((QW2, KNN), jnp.int32),
          pltpu.SemaphoreType.DMA,
      ],
  )
  def sc1(x0_h, s0_h, q1_h, x1_h, s1_h, q2_h, g1_h, bb1_h,
          f1_h, i2_h,
          xv, yv, zv, sv, qxv, qyv, qzv, bbv, iall, rows, obuf, i2buf, sem):
    wid = lax.axis_index("s") * NC + lax.axis_index("c")

    pltpu.sync_copy(x0_h.at[0], xv)
    pltpu.sync_copy(x0_h.at[1], yv)
    pltpu.sync_copy(x0_h.at[2], zv)
    pltpu.sync_copy(s0_h.at[0], sv)
    qbase = wid * QW1 * L
    pltpu.sync_copy(q1_h.at[0, pl.ds(qbase, QW1 * L)], qxv)
    pltpu.sync_copy(q1_h.at[1, pl.ds(qbase, QW1 * L)], qyv)
    pltpu.sync_copy(q1_h.at[2, pl.ds(qbase, QW1 * L)], qzv)
    pltpu.sync_copy(bb1_h.at[pl.ds(wid * QW1, QW1)], bbv)

    def q1_body(q, _):
      qx = qxv[pl.ds(q * L, L)]
      qy = qyv[pl.ds(q * L, L)]
      qz = qzv[pl.ds(q * L, L)]
      rd, ri = _knn_scan(qx, qy, qz, xv, yv, zv, sv, C1)
      iall[pl.ds(q * KNN, KNN)] = ri
      return 0

    lax.fori_loop(0, QW1, q1_body, 0)

    copies = []
    for g in range(QW1 * KNN // 128):
      copies.append(pltpu.async_copy(
          g1_h.at[iall.at[pl.ds(g * 128, 128)]],
          rows.at[pl.ds(g * 128, 128)], sem))
    for cp in copies:
      cp.wait()

    def q1_max(q, _):
      neg = jnp.full((L,), -jnp.inf, jnp.float32)

      def gm(j, acc):
        a0, a1 = acc
        return (jnp.maximum(a0, rows[q * KNN + j, pl.ds(0, L)]),
                jnp.maximum(a1, rows[q * KNN + j, pl.ds(L, L)]))

      a0, a1 = lax.fori_loop(0, KNN, gm, (neg, neg))
      obuf[q, pl.ds(0, L)] = jnp.maximum(a0 + bbv[q, pl.ds(0, L)], 0.0)
      obuf[q, pl.ds(L, L)] = jnp.maximum(a1 + bbv[q, pl.ds(L, L)], 0.0)
      return 0

    lax.fori_loop(0, QW1, q1_max, 0)
    pltpu.sync_copy(obuf, f1_h.at[pl.ds(wid * QW1, QW1)])

    pltpu.sync_copy(x1_h.at[0], xv.at[pl.ds(0, N1)])
    pltpu.sync_copy(x1_h.at[1], yv.at[pl.ds(0, N1)])
    pltpu.sync_copy(x1_h.at[2], zv.at[pl.ds(0, N1)])
    pltpu.sync_copy(s1_h.at[0], sv.at[pl.ds(0, N1)])
    q2base = wid * QW2 * L
    pltpu.sync_copy(q2_h.at[0, pl.ds(q2base, QW2 * L)], qxv.at[pl.ds(0, QW2 * L)])
    pltpu.sync_copy(q2_h.at[1, pl.ds(q2base, QW2 * L)], qyv.at[pl.ds(0, QW2 * L)])
    pltpu.sync_copy(q2_h.at[2, pl.ds(q2base, QW2 * L)], qzv.at[pl.ds(0, QW2 * L)])

    def q2_body(q, _):
      qx = qxv[pl.ds(q * L, L)]
      qy = qyv[pl.ds(q * L, L)]
      qz = qzv[pl.ds(q * L, L)]
      rd, ri = _knn_scan(qx, qy, qz, xv, yv, zv, sv, C2)
      i2buf[q, pl.ds(0, KNN)] = ri
      return 0

    lax.fori_loop(0, QW2, q2_body, 0)
    pltpu.sync_copy(i2buf, i2_h.at[pl.ds(wid * QW2, QW2)])

  return sc1


def _make_sc2():
  mesh = plsc.VectorSubcoreMesh(core_axis_name="c", subcore_axis_name="s",
                                num_cores=NC, num_subcores=NS)

  @functools.partial(
      pl.kernel,
      out_type=[jax.ShapeDtypeStruct((N2, O2), jnp.float32)],
      mesh=mesh,
      compiler_params=pltpu.CompilerParams(
          needs_layout_passes=False, use_tc_tiling_on_sc=False),
      scratch_types=[
          pltpu.VMEM((QW2 * KNN // 128, 128), jnp.int32),
          pltpu.VMEM((QW2 * KNN, O2), jnp.float32),
          pltpu.VMEM((QW2, O2), jnp.float32),
          pltpu.VMEM((QW2, O2), jnp.float32),
          pltpu.SemaphoreType.DMA,
      ],
  )
  def sc2(i2r_h, g2_h, bb2_h, f2_h, iall, rows, bbv, obuf, sem):
    wid = lax.axis_index("s") * NC + lax.axis_index("c")
    pltpu.sync_copy(
        i2r_h.at[pl.ds(wid * (QW2 * KNN // 128), QW2 * KNN // 128)], iall)
    pltpu.sync_copy(bb2_h.at[pl.ds(wid * QW2, QW2)], bbv)
    copies = []
    for g in range(QW2 * KNN // 128):
      copies.append(pltpu.async_copy(
          g2_h.at[iall.at[g]], rows.at[pl.ds(g * 128, 128)], sem))
    for cp in copies:
      cp.wait()

    def q_max(q, _):
      neg = jnp.full((L,), -jnp.inf, jnp.float32)

      def gm(j, acc):
        return tuple(
            jnp.maximum(acc[h], rows[q * KNN + j, pl.ds(h * L, L)])
            for h in range(O2 // L))

      acc = lax.fori_loop(0, KNN, gm, (neg,) * (O2 // L))
      for h in range(O2 // L):
        obuf[q, pl.ds(h * L, L)] = jnp.maximum(
            acc[h] + bbv[q, pl.ds(h * L, L)], 0.0)
      return 0

    lax.fori_loop(0, QW2, q_max, 0)
    pltpu.sync_copy(obuf, f2_h.at[pl.ds(wid * QW2, QW2)])

  return sc2


def kernel(xyzs_0, xyzs_1, xyzs_2,
           l0_W1, l0_b1, l0_W2, l0_b2,
           m0_W1, m0_b1, m0_W2, m0_b2,
           c0_W, c0_b,
           m1_W1, m1_b1, m1_W2, m1_b2,
           c1_W, c1_b):
  f32 = jnp.float32
  x0 = xyzs_0.reshape(3, N0)
  x1 = xyzs_1.reshape(3, N1)
  x2 = xyzs_2.reshape(3, N2)

  tc1 = pl.pallas_call(
      _tc1_body,
      out_shape=[
          jax.ShapeDtypeStruct((16, N0), f32),
          jax.ShapeDtypeStruct((O1, N0), f32),
          jax.ShapeDtypeStruct((O1, N1), f32),
          jax.ShapeDtypeStruct((1, N0), f32),
          jax.ShapeDtypeStruct((1, N1), f32),
      ],
  )
  feat0, g1, bb1, sq0, sq1 = tc1(
      x0, x1,
      l0_W1, l0_b1.reshape(16, 1), l0_W2, l0_b2.reshape(16, 1),
      m0_W1, m0_b1.reshape(16, 1), m0_W2, m0_b2.reshape(32, 1),
      c0_W, c0_b.reshape(O1, 1))

  xb0 = lax.optimization_barrier(x0.astype(jnp.bfloat16)).astype(f32)
  xb1 = lax.optimization_barrier(x1.astype(jnp.bfloat16)).astype(f32)
  xb2 = lax.optimization_barrier(x2.astype(jnp.bfloat16)).astype(f32)

  q1rep = jnp.broadcast_to(xb1[:, :, None], (3, N1, L)).reshape(3, N1 * L)
  q2rep = jnp.broadcast_to(xb2[:, :, None], (3, N2, L)).reshape(3, N2 * L)

  sc1 = _make_sc1()
  feat1_t, idx2 = sc1(xb0, sq0, q1rep, xb1, sq1, q2rep,
                      g1.T.reshape(N0, O1), bb1.T.reshape(N1, O1))
  feat1 = feat1_t.T.reshape(O1, N1)

  tc2 = pl.pallas_call(
      _tc2_body,
      out_shape=[
          jax.ShapeDtypeStruct((O2, N1), f32),
          jax.ShapeDtypeStruct((O2, N2), f32),
      ],
  )
  g2, bb2 = tc2(feat1, x1, x2,
                m1_W1, m1_b1.reshape(32, 1), m1_W2, m1_b2.reshape(O2, 1),
                c1_W, c1_b.reshape(O2, 1))

  sc2 = _make_sc2()
  idx2r = idx2.reshape(N2 * KNN // 128, 128)
  (feat2_t,) = sc2(idx2r, g2.T.reshape(N1, O2), bb2.T.reshape(N2, O2))
  feat2 = feat2_t.T.reshape(O2, N2)

  return (feat0.reshape(1, 16, N0),
          feat1.reshape(1, O1, N1),
          feat2.reshape(1, O2, N2))


# speedup vs baseline: 28.4707x; 6.7655x over previous
import functools

import jax
import jax.numpy as jnp
from jax import lax
from jax.experimental import pallas as pl
from jax.experimental.pallas import tpu as pltpu
from jax.experimental.pallas import tpu_sc as plsc

N0, N1, N2 = 8192, 2048, 512
KNN = 16
L = 16
NC, NS = 2, 16
NW = NC * NS
QW1 = N1 // NW
QW2 = N2 // NW
C1 = N0 // L
C2 = N1 // L
O1, O2 = 32, 64


def _tc1_body(x0_r, x1_r, l0W1_r, l0b1_r, l0W2_r, l0b2_r,
              m0W1_r, m0b1_r, m0W2_r, m0b2_r, c0W_r, c0b_r,
              feat0_o, g1_o, bb1_o, sq0_o, sq1_o):
  x0 = x0_r[...]
  x1 = x1_r[...]
  dot = functools.partial(jnp.dot, preferred_element_type=jnp.float32)
  h = jnp.maximum(dot(l0W1_r[...], x0) + l0b1_r[...], 0.0)
  feat0 = jnp.maximum(dot(l0W2_r[...], h) + l0b2_r[...], 0.0)
  feat0_o[...] = feat0
  h = jnp.maximum(dot(m0W1_r[...], feat0) + m0b1_r[...], 0.0)
  f = jnp.maximum(dot(m0W2_r[...], h) + m0b2_r[...], 0.0)
  c0W = c0W_r[...]
  g1_o[...] = dot(c0W[:, :3], x0) + dot(c0W[:, 3:], f)
  bb1_o[...] = c0b_r[...] - dot(c0W[:, :3], x1)
  sq0_o[...] = jnp.sum(x0 * x0, axis=0, keepdims=True)
  sq1_o[...] = jnp.sum(x1 * x1, axis=0, keepdims=True)


def _tc2_body(feat1_r, x1_r, x2_r, m1W1_r, m1b1_r, m1W2_r, m1b2_r,
              c1W_r, c1b_r, g2_o, bb2_o):
  dot = functools.partial(jnp.dot, preferred_element_type=jnp.float32)
  feat1 = feat1_r[...]
  h = jnp.maximum(dot(m1W1_r[...], feat1) + m1b1_r[...], 0.0)
  f2 = jnp.maximum(dot(m1W2_r[...], h) + m1b2_r[...], 0.0)
  c1W = c1W_r[...]
  g2_o[...] = dot(c1W[:, :3], x1_r[...]) + dot(c1W[:, 3:], f2)
  bb2_o[...] = c1b_r[...] - dot(c1W[:, :3], x2_r[...])


def _merge16(rd, ri, d, ji):
  cd, ci = plsc.sort_key_val(d, ji, descending=True)
  m = cd < rd
  nd = jnp.where(m, cd, rd)
  ni = jnp.where(m, ci, ri)
  nd2, ni2 = plsc.sort_key_val(nd, ni)
  return nd2, ni2


def _knn4(qs, xv, yv, zv, sv, nb, cmbuf):
  nq = len(qs)

  def blk(b, _):
    base = b * 256
    mvs = [None] * nq
    for v in range(16):
      x = xv[pl.ds(base + v * L, L)]
      y = yv[pl.ds(base + v * L, L)]
      z = zv[pl.ds(base + v * L, L)]
      s = sv[pl.ds(base + v * L, L)]
      for qi, (qx, qy, qz) in enumerate(qs):
        d = s - 2.0 * (qx * x + qy * y + qz * z)
        mvs[qi] = d if v == 0 else jnp.minimum(mvs[qi], d)
    for qi in range(nq):
      cmbuf[pl.ds(qi * (nb * L) + b * L, L)] = mvs[qi]
    return 0

  lax.fori_loop(0, nb, blk, 0)

  inf = jnp.full((L,), jnp.inf, jnp.float32)
  zero = jnp.zeros((L,), jnp.int32)

  def pb(b, carry):
    rds, ris = list(carry[:nq]), list(carry[nq:])
    ids0 = lax.iota(jnp.int32, L) + b * L
    for qi in range(nq):
      cmv = cmbuf[pl.ds(qi * (nb * L) + b * L, L)]
      rds[qi], ris[qi] = _merge16(rds[qi], ris[qi], cmv, ids0)
    return tuple(rds) + tuple(ris)

  res = lax.fori_loop(0, nb, pb, (inf,) * nq + (zero,) * nq)
  cri = res[nq:]

  sixteen = lax.iota(jnp.int32, L) * 16
  rds = [inf] * nq
  ris = [zero] * nq
  for k in range(16):
    for qi, (qx, qy, qz) in enumerate(qs):
      c = cri[qi][k]
      pos0 = ((c >> 4) << 8) + (c & 15)
      idxv = sixteen + pos0
      gx = plsc.load_gather(xv, [idxv])
      gy = plsc.load_gather(yv, [idxv])
      gz = plsc.load_gather(zv, [idxv])
      gs = plsc.load_gather(sv, [idxv])
      d = gs - 2.0 * (qx * gx + qy * gy + qz * gz)
      rds[qi], ris[qi] = _merge16(rds[qi], ris[qi], d, idxv)
  return rds, ris


def _make_sc1():
  mesh = plsc.VectorSubcoreMesh(core_axis_name="c", subcore_axis_name="s",
                                num_cores=NC, num_subcores=NS)

  @functools.partial(
      pl.kernel,
      out_type=[
          jax.ShapeDtypeStruct((N1, O1), jnp.float32),
          jax.ShapeDtypeStruct((N2, KNN), jnp.int32),
      ],
      mesh=mesh,
      compiler_params=pltpu.CompilerParams(
          needs_layout_passes=False, use_tc_tiling_on_sc=False),
      scratch_types=[
          pltpu.VMEM((N0,), jnp.float32),
          pltpu.VMEM((N0,), jnp.float32),
          pltpu.VMEM((N0,), jnp.float32),
          pltpu.VMEM((N0,), jnp.float32),
          pltpu.VMEM((QW1 * L,), jnp.float32),
          pltpu.VMEM((QW1 * L,), jnp.float32),
          pltpu.VMEM((QW1 * L,), jnp.float32),
          pltpu.VMEM((QW1, O1), jnp.float32),
          pltpu.VMEM((QW1 * KNN,), jnp.int32),
          pltpu.VMEM((QW1 * KNN, O1), jnp.float32),
          pltpu.VMEM((QW1, O1), jnp.float32),
          pltpu.VMEM((QW2, KNN), jnp.int32),
          pltpu.VMEM((4 * C1,), jnp.float32),
          pltpu.SemaphoreType.DMA,
      ],
  )
  def sc1(x0_h, s0_h, q1_h, x1_h, s1_h, q2_h, g1_h, bb1_h,
          f1_h, i2_h,
          xv, yv, zv, sv, qxv, qyv, qzv, bbv, iall, rows, obuf, i2buf,
          cmbuf, sem):
    wid = lax.axis_index("s") * NC + lax.axis_index("c")

    pltpu.sync_copy(x0_h.at[0], xv)
    pltpu.sync_copy(x0_h.at[1], yv)
    pltpu.sync_copy(x0_h.at[2], zv)
    pltpu.sync_copy(s0_h.at[0], sv)
    qbase = wid * QW1 * L
    pltpu.sync_copy(q1_h.at[0, pl.ds(qbase, QW1 * L)], qxv)
    pltpu.sync_copy(q1_h.at[1, pl.ds(qbase, QW1 * L)], qyv)
    pltpu.sync_copy(q1_h.at[2, pl.ds(qbase, QW1 * L)], qzv)
    pltpu.sync_copy(bb1_h.at[pl.ds(wid * QW1, QW1)], bbv)

    def q1_group(g, _):
      qs = []
      for qi in range(4):
        qoff = (g * 4 + qi) * L
        qs.append((qxv[pl.ds(qoff, L)], qyv[pl.ds(qoff, L)],
                   qzv[pl.ds(qoff, L)]))
      _, ris = _knn4(qs, xv, yv, zv, sv, N0 // 256, cmbuf)
      for qi in range(4):
        iall[pl.ds((g * 4 + qi) * KNN, KNN)] = ris[qi]
      return 0

    lax.fori_loop(0, QW1 // 4, q1_group, 0)

    copies = []
    for g in range(QW1 * KNN // 128):
      copies.append(pltpu.async_copy(
          g1_h.at[iall.at[pl.ds(g * 128, 128)]],
          rows.at[pl.ds(g * 128, 128)], sem))
    for cp in copies:
      cp.wait()

    def q1_max(q, _):
      neg = jnp.full((L,), -jnp.inf, jnp.float32)

      def gm(j, acc):
        a0, a1 = acc
        return (jnp.maximum(a0, rows[q * KNN + j, pl.ds(0, L)]),
                jnp.maximum(a1, rows[q * KNN + j, pl.ds(L, L)]))

      a0, a1 = lax.fori_loop(0, KNN, gm, (neg, neg))
      obuf[q, pl.ds(0, L)] = jnp.maximum(a0 + bbv[q, pl.ds(0, L)], 0.0)
      obuf[q, pl.ds(L, L)] = jnp.maximum(a1 + bbv[q, pl.ds(L, L)], 0.0)
      return 0

    lax.fori_loop(0, QW1, q1_max, 0)
    pltpu.sync_copy(obuf, f1_h.at[pl.ds(wid * QW1, QW1)])

    pltpu.sync_copy(x1_h.at[0], xv.at[pl.ds(0, N1)])
    pltpu.sync_copy(x1_h.at[1], yv.at[pl.ds(0, N1)])
    pltpu.sync_copy(x1_h.at[2], zv.at[pl.ds(0, N1)])
    pltpu.sync_copy(s1_h.at[0], sv.at[pl.ds(0, N1)])
    q2base = wid * QW2 * L
    pltpu.sync_copy(q2_h.at[0, pl.ds(q2base, QW2 * L)], qxv.at[pl.ds(0, QW2 * L)])
    pltpu.sync_copy(q2_h.at[1, pl.ds(q2base, QW2 * L)], qyv.at[pl.ds(0, QW2 * L)])
    pltpu.sync_copy(q2_h.at[2, pl.ds(q2base, QW2 * L)], qzv.at[pl.ds(0, QW2 * L)])

    def q2_group(g, _):
      qs = []
      for qi in range(4):
        qoff = (g * 4 + qi) * L
        qs.append((qxv[pl.ds(qoff, L)], qyv[pl.ds(qoff, L)],
                   qzv[pl.ds(qoff, L)]))
      _, ris = _knn4(qs, xv, yv, zv, sv, N1 // 256, cmbuf)
      for qi in range(4):
        i2buf[g * 4 + qi, pl.ds(0, KNN)] = ris[qi]
      return 0

    lax.fori_loop(0, QW2 // 4, q2_group, 0)
    pltpu.sync_copy(i2buf, i2_h.at[pl.ds(wid * QW2, QW2)])

  return sc1


def _make_sc2():
  mesh = plsc.VectorSubcoreMesh(core_axis_name="c", subcore_axis_name="s",
                                num_cores=NC, num_subcores=NS)

  @functools.partial(
      pl.kernel,
      out_type=[jax.ShapeDtypeStruct((N2, O2), jnp.float32)],
      mesh=mesh,
      compiler_params=pltpu.CompilerParams(
          needs_layout_passes=False, use_tc_tiling_on_sc=False),
      scratch_types=[
          pltpu.VMEM((QW2 * KNN // 128, 128), jnp.int32),
          pltpu.VMEM((QW2 * KNN, O2), jnp.float32),
          pltpu.VMEM((QW2, O2), jnp.float32),
          pltpu.VMEM((QW2, O2), jnp.float32),
          pltpu.SemaphoreType.DMA,
      ],
  )
  def sc2(i2r_h, g2_h, bb2_h, f2_h, iall, rows, bbv, obuf, sem):
    wid = lax.axis_index("s") * NC + lax.axis_index("c")
    pltpu.sync_copy(
        i2r_h.at[pl.ds(wid * (QW2 * KNN // 128), QW2 * KNN // 128)], iall)
    pltpu.sync_copy(bb2_h.at[pl.ds(wid * QW2, QW2)], bbv)
    copies = []
    for g in range(QW2 * KNN // 128):
      copies.append(pltpu.async_copy(
          g2_h.at[iall.at[g]], rows.at[pl.ds(g * 128, 128)], sem))
    for cp in copies:
      cp.wait()

    def q_max(q, _):
      neg = jnp.full((L,), -jnp.inf, jnp.float32)

      def gm(j, acc):
        return tuple(
            jnp.maximum(acc[h], rows[q * KNN + j, pl.ds(h * L, L)])
            for h in range(O2 // L))

      acc = lax.fori_loop(0, KNN, gm, (neg,) * (O2 // L))
      for h in range(O2 // L):
        obuf[q, pl.ds(h * L, L)] = jnp.maximum(
            acc[h] + bbv[q, pl.ds(h * L, L)], 0.0)
      return 0

    lax.fori_loop(0, QW2, q_max, 0)
    pltpu.sync_copy(obuf, f2_h.at[pl.ds(wid * QW2, QW2)])

  return sc2


def kernel(xyzs_0, xyzs_1, xyzs_2,
           l0_W1, l0_b1, l0_W2, l0_b2,
           m0_W1, m0_b1, m0_W2, m0_b2,
           c0_W, c0_b,
           m1_W1, m1_b1, m1_W2, m1_b2,
           c1_W, c1_b):
  f32 = jnp.float32
  x0 = xyzs_0.reshape(3, N0)
  x1 = xyzs_1.reshape(3, N1)
  x2 = xyzs_2.reshape(3, N2)

  tc1 = pl.pallas_call(
      _tc1_body,
      out_shape=[
          jax.ShapeDtypeStruct((16, N0), f32),
          jax.ShapeDtypeStruct((O1, N0), f32),
          jax.ShapeDtypeStruct((O1, N1), f32),
          jax.ShapeDtypeStruct((1, N0), f32),
          jax.ShapeDtypeStruct((1, N1), f32),
      ],
  )
  feat0, g1, bb1, sq0, sq1 = tc1(
      x0, x1,
      l0_W1, l0_b1.reshape(16, 1), l0_W2, l0_b2.reshape(16, 1),
      m0_W1, m0_b1.reshape(16, 1), m0_W2, m0_b2.reshape(32, 1),
      c0_W, c0_b.reshape(O1, 1))

  xb0 = lax.optimization_barrier(x0.astype(jnp.bfloat16)).astype(f32)
  xb1 = lax.optimization_barrier(x1.astype(jnp.bfloat16)).astype(f32)
  xb2 = lax.optimization_barrier(x2.astype(jnp.bfloat16)).astype(f32)

  q1rep = jnp.broadcast_to(xb1[:, :, None], (3, N1, L)).reshape(3, N1 * L)
  q2rep = jnp.broadcast_to(xb2[:, :, None], (3, N2, L)).reshape(3, N2 * L)

  sc1 = _make_sc1()
  feat1_t, idx2 = sc1(xb0, sq0, q1rep, xb1, sq1, q2rep,
                      g1.T.reshape(N0, O1), bb1.T.reshape(N1, O1))
  feat1 = feat1_t.T.reshape(O1, N1)

  tc2 = pl.pallas_call(
      _tc2_body,
      out_shape=[
          jax.ShapeDtypeStruct((O2, N1), f32),
          jax.ShapeDtypeStruct((O2, N2), f32),
      ],
  )
  g2, bb2 = tc2(feat1, x1, x2,
                m1_W1, m1_b1.reshape(32, 1), m1_W2, m1_b2.reshape(O2, 1),
                c1_W, c1_b.reshape(O2, 1))

  sc2 = _make_sc2()
  idx2r = idx2.reshape(N2 * KNN // 128, 128)
  (feat2_t,) = sc2(idx2r, g2.T.reshape(N1, O2), bb2.T.reshape(N2, O2))
  feat2 = feat2_t.T.reshape(O2, N2)

  return (feat0.reshape(1, 16, N0),
          feat1.reshape(1, O1, N1),
          feat2.reshape(1, O2, N2))
